# baseline (speedup 1.0000x reference)
import jax
import jax.numpy as jnp
from jax.experimental import pallas as pl

_B, _C, _D = 16384, 26, 8
_R = _C * _D


def _body(pat_ref, out_ref):
    out_ref[...] = jnp.broadcast_to(pat_ref[...], (_D, _B))


def _tc_broadcast(pat):
    return pl.pallas_call(
        _body,
        grid=(_C,),
        in_specs=[pl.BlockSpec((_D, 1), lambda i: (0, 0))],
        out_specs=pl.BlockSpec((_D, _B), lambda i: (i, 0)),
        out_shape=jax.ShapeDtypeStruct((_R, _B), jnp.float32),
    )(pat)


def kernel(input, table):
    del input
    pat = table.reshape(_D, 1)
    out2d = _tc_broadcast(pat)
    return out2d.reshape(_C, _D, _B).transpose(2, 0, 1)


# speedup vs baseline: 1.9021x; 1.9021x over previous
import jax
import jax.numpy as jnp
from jax.experimental import pallas as pl

_B, _C, _D = 16384, 26, 8
_R = _C * _D
_BLK = 4096
_GRID = _B // _BLK


def _body(pat_ref, out_ref):
    out_ref[...] = jnp.broadcast_to(pat_ref[...], (_R, _BLK))


def _tc_broadcast(pat):
    return pl.pallas_call(
        _body,
        grid=(_GRID,),
        in_specs=[pl.BlockSpec((_R, 1), lambda i: (0, 0))],
        out_specs=pl.BlockSpec((_R, _BLK), lambda i: (0, i)),
        out_shape=jax.ShapeDtypeStruct((_R, _B), jnp.float32),
    )(pat)


def kernel(input, table):
    del input
    pat = jnp.tile(table.reshape(-1), _C)[:, None]
    out2d = _tc_broadcast(pat)
    return out2d.reshape(_C, _D, _B).transpose(2, 0, 1)
